# Initial kernel scaffold; baseline (speedup 1.0000x reference)
import functools

import jax
import jax.numpy as jnp
from jax import lax
from jax.experimental import pallas as pl
from jax.experimental.pallas import tpu as pltpu
from jax.experimental.pallas import tpu_sc as plsc

NC = 2
NS = 16
NW = NC * NS
LANES = 16
CHUNK = 128


def _cdiv(a, b):
    return (a + b - 1) // b


def _deg_kernel(nt, epw, nw):
    mesh = plsc.VectorSubcoreMesh(core_axis_name="c", subcore_axis_name="s")

    @functools.partial(
        pl.kernel,
        out_type=jax.ShapeDtypeStruct((nw, nt), jnp.float32),
        mesh=mesh,
        scratch_types=[
            pltpu.VMEM((epw,), jnp.int32),
            pltpu.VMEM((nt,), jnp.float32),
        ],
    )
    def degk(dflat_hbm, hist_hbm, idx_v, hist_v):
        cid = lax.axis_index("c")
        sid = lax.axis_index("s")
        wid = sid * NC + cid
        pltpu.sync_copy(dflat_hbm.at[pl.ds(wid * epw, epw)], idx_v)

        def zbody(i, carry):
            hist_v[pl.ds(i * LANES, LANES)] = jnp.zeros((LANES,), jnp.float32)
            return carry

        lax.fori_loop(0, nt // LANES, zbody, 0)
        ones = jnp.ones((LANES,), jnp.float32)

        def ebody(w, carry):
            idx = idx_v[pl.ds(w * LANES, LANES)]
            plsc.addupdate_scatter(hist_v, [idx], ones)
            return carry

        lax.fori_loop(0, epw // LANES, ebody, 0)
        pltpu.sync_copy(hist_v, hist_hbm.at[wid])

    return degk


def _round_kernel(n, fo, nt, nch):
    mesh = plsc.VectorSubcoreMesh(core_axis_name="c", subcore_axis_name="s")
    rp = n // NS
    zp = nt // NS

    @functools.partial(
        pl.kernel,
        out_type=jax.ShapeDtypeStruct((NC, nt, fo), jnp.float32),
        mesh=mesh,
        scratch_types=[
            pltpu.VMEM((nch, CHUNK), jnp.int32),
            pltpu.VMEM((nch, CHUNK), jnp.int32),
            pltpu.VMEM((2, CHUNK, fo), jnp.float32),
            pltpu.VMEM_SHARED((n, fo), jnp.float32),
            pltpu.VMEM_SHARED((nt, fo), jnp.float32),
            pltpu.SemaphoreType.DMA,
            pltpu.SemaphoreType.DMA,
        ],
    )
    def roundk(p_hbm, s_hbm, d_hbm, z_hbm, out_hbm,
               idx_s, idx_d, rows, sp_p, sp_t, sem0, sem1):
        cid = lax.axis_index("c")
        sid = lax.axis_index("s")
        wid = sid * NC + cid
        pltpu.sync_copy(p_hbm.at[pl.ds(sid * rp, rp)], sp_p.at[pl.ds(sid * rp, rp)])
        pltpu.sync_copy(z_hbm, sp_t.at[pl.ds(sid * zp, zp)])
        pltpu.sync_copy(s_hbm.at[wid], idx_s)
        pltpu.sync_copy(d_hbm.at[wid], idx_d)
        plsc.subcore_barrier()

        sems = (sem0, sem1)
        descs = [None, None]
        descs[0] = pltpu.async_copy(sp_p.at[idx_s.at[0]], rows.at[0], sem0)
        for j in range(nch):
            cur = j % 2
            nxt = (j + 1) % 2
            if j + 1 < nch:
                descs[nxt] = pltpu.async_copy(
                    sp_p.at[idx_s.at[j + 1]], rows.at[nxt], sems[nxt])
            descs[cur].wait()
            pltpu.sync_copy(rows.at[cur], sp_t.at[idx_d.at[j]], add=True)
        plsc.subcore_barrier()
        pltpu.sync_copy(sp_t.at[pl.ds(sid * zp, zp)],
                        out_hbm.at[cid].at[pl.ds(sid * zp, zp)])

    return roundk


def _tc_prep_body(x_ref, w_ref, hist_ref, p0_ref, dinv_ref, dinvs_ref, *, n):
    deg = jnp.sum(hist_ref[...], axis=0)[:n] + 1.0
    dinvs = lax.rsqrt(deg)
    dinv_ref[...] = 1.0 / deg
    dinvs_ref[...] = dinvs
    xw = lax.dot_general(x_ref[...], w_ref[...],
                         (((1,), (1,)), ((), ())),
                         preferred_element_type=jnp.float32)
    p0_ref[...] = xw * dinvs[:, None]


def _tc_combine_body(tp_ref, p_ref, scale_ref, bias_ref, out_ref, *, n):
    t = tp_ref[0, :n, :] + tp_ref[1, :n, :] + p_ref[...]
    out_ref[...] = scale_ref[...][:, None] * t + bias_ref[...][None, :]


def kernel(x, edge_index, W, b):
    n, f_in = x.shape
    fo = W.shape[0]
    e = edge_index.shape[1]

    nch = _cdiv(_cdiv(e, NW), CHUNK)
    epw = nch * CHUNK
    epad = NW * epw
    nt = _cdiv(n + 1, CHUNK * NS) * CHUNK * NS

    src = edge_index[0]
    dst = edge_index[1]
    pad = epad - e
    src_p = jnp.concatenate([src, jnp.zeros((pad,), jnp.int32)])
    dst_p = jnp.concatenate([dst, jnp.full((pad,), n, jnp.int32)])
    src3 = src_p.reshape(NW, nch, CHUNK)
    dst3 = dst_p.reshape(NW, nch, CHUNK)
    zrows = jnp.zeros((nt // NS, fo), jnp.float32)

    hist = _deg_kernel(nt, epw, NW)(dst_p)

    tc_prep = pl.pallas_call(
        functools.partial(_tc_prep_body, n=n),
        out_shape=(
            jax.ShapeDtypeStruct((n, fo), jnp.float32),
            jax.ShapeDtypeStruct((n,), jnp.float32),
            jax.ShapeDtypeStruct((n,), jnp.float32),
        ),
    )
    p0, dinv, dinvs = tc_prep(x, W, hist)

    tc_combine = pl.pallas_call(
        functools.partial(_tc_combine_body, n=n),
        out_shape=jax.ShapeDtypeStruct((n, fo), jnp.float32),
    )

    roundk = _round_kernel(n, fo, nt, nch)
    t1 = roundk(p0, src3, dst3, zrows)
    p1 = tc_combine(t1, p0, dinv, jnp.zeros((fo,), jnp.float32))
    t2 = roundk(t1 * 0 + t2_placeholder if False else p1, src3, dst3, zrows)
    out = tc_combine(t2, p1, dinvs, b)
    return out


# speedup vs baseline: 24.6183x; 24.6183x over previous
import functools

import jax
import jax.numpy as jnp
from jax import lax
from jax.experimental import pallas as pl
from jax.experimental.pallas import tpu as pltpu
from jax.experimental.pallas import tpu_sc as plsc

NC = 2
NS = 16
NW = NC * NS
LANES = 16
CHUNK = 128


def _cdiv(a, b):
    return (a + b - 1) // b


def _deg_kernel(nt, epw, nw):
    mesh = plsc.VectorSubcoreMesh(core_axis_name="c", subcore_axis_name="s")

    @functools.partial(
        pl.kernel,
        out_type=jax.ShapeDtypeStruct((nw, nt), jnp.float32),
        mesh=mesh,
        scratch_types=[
            pltpu.VMEM((epw,), jnp.int32),
            pltpu.VMEM((nt,), jnp.float32),
        ],
        compiler_params=pltpu.CompilerParams(needs_layout_passes=False),
    )
    def degk(dflat_hbm, hist_hbm, idx_v, hist_v):
        cid = lax.axis_index("c")
        sid = lax.axis_index("s")
        wid = sid * NC + cid
        pltpu.sync_copy(dflat_hbm.at[pl.ds(wid * epw, epw)], idx_v)

        def zbody(i, carry):
            hist_v[pl.ds(i * LANES, LANES)] = jnp.zeros((LANES,), jnp.float32)
            return carry

        lax.fori_loop(0, nt // LANES, zbody, 0)
        ones = jnp.ones((LANES,), jnp.float32)

        def ebody(w, carry):
            idx = idx_v[pl.ds(w * LANES, LANES)]
            plsc.addupdate_scatter(hist_v, [idx], ones)
            return carry

        lax.fori_loop(0, epw // LANES, ebody, 0)
        pltpu.sync_copy(hist_v, hist_hbm.at[wid])

    return degk


def _round_kernel(n16, fo, nch):
    mesh = plsc.VectorSubcoreMesh(core_axis_name="c", subcore_axis_name="s")
    zp = n16 // NS

    @functools.partial(
        pl.kernel,
        out_type=jax.ShapeDtypeStruct((NC, n16, fo), jnp.float32),
        mesh=mesh,
        scratch_types=[
            pltpu.VMEM((nch, CHUNK), jnp.int32),
            pltpu.VMEM((nch, CHUNK), jnp.int32),
            pltpu.VMEM((2, CHUNK, fo), jnp.float32),
            pltpu.VMEM_SHARED((n16, fo), jnp.float32),
            pltpu.SemaphoreType.DMA,
            pltpu.SemaphoreType.DMA,
        ],
        compiler_params=pltpu.CompilerParams(
            needs_layout_passes=False, use_tc_tiling_on_sc=False),
    )
    def roundk(p_hbm, s_hbm, d_hbm, z_hbm, out_hbm,
               idx_s, idx_d, rows, sp_t, sem0, sem1):
        cid = lax.axis_index("c")
        sid = lax.axis_index("s")
        wid = sid * NC + cid
        pltpu.sync_copy(z_hbm, sp_t.at[pl.ds(sid * zp, zp)])
        pltpu.sync_copy(s_hbm.at[wid], idx_s)
        pltpu.sync_copy(d_hbm.at[wid], idx_d)
        plsc.subcore_barrier()

        sems = (sem0, sem1)
        descs = [None, None]
        descs[0] = pltpu.async_copy(p_hbm.at[idx_s.at[0]], rows.at[0], sem0)
        for j in range(nch):
            cur = j % 2
            nxt = (j + 1) % 2
            if j + 1 < nch:
                descs[nxt] = pltpu.async_copy(
                    p_hbm.at[idx_s.at[j + 1]], rows.at[nxt], sems[nxt])
            descs[cur].wait()
            pltpu.sync_copy(rows.at[cur], sp_t.at[idx_d.at[j]], add=True)
        plsc.subcore_barrier()
        pltpu.sync_copy(sp_t.at[pl.ds(sid * zp, zp)],
                        out_hbm.at[cid].at[pl.ds(sid * zp, zp)])

    return roundk


def _tc_prep_body(x_ref, w_ref, hist_ref, p0_ref, dinv_ref, dinvs_ref, *, n, n16):
    deg = jnp.sum(hist_ref[...], axis=0)[:n] + 1.0
    dinvs = lax.rsqrt(deg)
    dinv_ref[...] = 1.0 / deg
    dinvs_ref[...] = dinvs
    xw = lax.dot_general(x_ref[...], w_ref[...],
                         (((1,), (1,)), ((), ())),
                         preferred_element_type=jnp.float32)
    p0_ref[:n, :] = xw * dinvs[:, None]
    if n16 > n:
        p0_ref[n:, :] = jnp.zeros((n16 - n, xw.shape[1]), jnp.float32)


def _tc_combine_body(tp_ref, p_ref, scale_ref, bias_ref, out_ref, *, n, n16):
    t = tp_ref[0, :n, :] + tp_ref[1, :n, :] + p_ref[:n, :]
    out_ref[:n, :] = scale_ref[...][:, None] * t + bias_ref[...][None, :]
    if n16 > n:
        out_ref[n:, :] = jnp.zeros((n16 - n, t.shape[1]), jnp.float32)


def kernel(x, edge_index, W, b):
    n, f_in = x.shape
    fo = W.shape[0]
    e = edge_index.shape[1]

    nch = _cdiv(_cdiv(e, NW), CHUNK)
    epw = nch * CHUNK
    epad = NW * epw
    n16 = NS * 8 * _cdiv(n + 1, NS * 8)

    src = edge_index[0]
    dst = edge_index[1]
    pad = epad - e
    src_p = jnp.concatenate([src, jnp.zeros((pad,), jnp.int32)])
    dst_p = jnp.concatenate([dst, jnp.full((pad,), n, jnp.int32)])
    src3 = src_p.reshape(NW, nch, CHUNK)
    dst3 = dst_p.reshape(NW, nch, CHUNK)
    zrows = jnp.zeros((n16 // NS, fo), jnp.float32)

    hist = _deg_kernel(n16, epw, NW)(dst_p)

    tc_prep = pl.pallas_call(
        functools.partial(_tc_prep_body, n=n, n16=n16),
        out_shape=(
            jax.ShapeDtypeStruct((n16, fo), jnp.float32),
            jax.ShapeDtypeStruct((n,), jnp.float32),
            jax.ShapeDtypeStruct((n,), jnp.float32),
        ),
    )
    p0, dinv, dinvs = tc_prep(x, W, hist)

    tc_combine = pl.pallas_call(
        functools.partial(_tc_combine_body, n=n, n16=n16),
        out_shape=jax.ShapeDtypeStruct((n16, fo), jnp.float32),
    )

    roundk = _round_kernel(n16, fo, nch)
    t1 = roundk(p0, src3, dst3, zrows)
    p1 = tc_combine(t1, p0, dinv, jnp.zeros((fo,), jnp.float32))
    t2 = roundk(p1, src3, dst3, zrows)
    out = tc_combine(t2, p1, dinvs, b)
    return out[:n]


# speedup vs baseline: 39.0207x; 1.5850x over previous
import functools

import jax
import jax.numpy as jnp
from jax import lax
from jax.experimental import pallas as pl
from jax.experimental.pallas import tpu as pltpu
from jax.experimental.pallas import tpu_sc as plsc

NC = 2
NS = 16
NW = NC * NS
LANES = 16
CHUNK = 128
NBUF = 8


def _cdiv(a, b):
    return (a + b - 1) // b


def _deg_kernel(nt, epw, nw):
    mesh = plsc.VectorSubcoreMesh(core_axis_name="c", subcore_axis_name="s")

    @functools.partial(
        pl.kernel,
        out_type=jax.ShapeDtypeStruct((nw, nt), jnp.float32),
        mesh=mesh,
        scratch_types=[
            pltpu.VMEM((epw,), jnp.int32),
            pltpu.VMEM((nt,), jnp.float32),
        ],
        compiler_params=pltpu.CompilerParams(needs_layout_passes=False),
    )
    def degk(dflat_hbm, hist_hbm, idx_v, hist_v):
        cid = lax.axis_index("c")
        sid = lax.axis_index("s")
        wid = sid * NC + cid
        pltpu.sync_copy(dflat_hbm.at[pl.ds(wid * epw, epw)], idx_v)

        def zbody(i, carry):
            hist_v[pl.ds(i * LANES, LANES)] = jnp.zeros((LANES,), jnp.float32)
            return carry

        lax.fori_loop(0, nt // LANES, zbody, 0)
        ones = jnp.ones((LANES,), jnp.float32)

        def ebody(w, carry):
            idx = idx_v[pl.ds(w * LANES, LANES)]
            plsc.addupdate_scatter(hist_v, [idx], ones)
            return carry

        lax.fori_loop(0, epw // LANES, ebody, 0)
        pltpu.sync_copy(hist_v, hist_hbm.at[wid])

    return degk


def _main_kernel(n16, fh, nch):
    mesh = plsc.VectorSubcoreMesh(core_axis_name="c", subcore_axis_name="s")
    rp = n16 // NS
    CR = rp // 4

    @functools.partial(
        pl.kernel,
        out_type=jax.ShapeDtypeStruct((n16, 2 * fh), jnp.float32),
        mesh=mesh,
        scratch_types=[
            pltpu.VMEM((nch, CHUNK), jnp.int32),
            pltpu.VMEM((nch, CHUNK), jnp.int32),
            pltpu.VMEM((NBUF, CHUNK, fh), jnp.float32),
            pltpu.VMEM((CR, fh), jnp.float32),
            pltpu.VMEM((CR, fh), jnp.float32),
            pltpu.VMEM((rp,), jnp.float32),
            pltpu.VMEM((rp,), jnp.float32),
            pltpu.VMEM((2 * fh,), jnp.float32),
            pltpu.VMEM_SHARED((n16, fh), jnp.float32),
            pltpu.VMEM_SHARED((n16, fh), jnp.float32),
            pltpu.SemaphoreType.DMA((NBUF,)),
            pltpu.SemaphoreType.DMA((NBUF,)),
        ],
        compiler_params=pltpu.CompilerParams(
            needs_layout_passes=False, use_tc_tiling_on_sc=False),
    )
    def maink(p0_hbm, s_hbm, d_hbm, z_hbm, dinv_hbm, dinvs_hbm, b_hbm, out_hbm,
              idx_s, idx_d, rows, cbA, cbB, dv, dv2, bv, sp_p, sp_t,
              gsem, ssem):
        cid = lax.axis_index("c")
        sid = lax.axis_index("s")
        r0 = sid * rp

        pltpu.sync_copy(p0_hbm.at[cid].at[pl.ds(r0, rp)], sp_p.at[pl.ds(r0, rp)])
        pltpu.sync_copy(z_hbm, sp_t.at[pl.ds(r0, rp)])
        pltpu.sync_copy(s_hbm.at[sid], idx_s)
        pltpu.sync_copy(d_hbm.at[sid], idx_d)
        pltpu.sync_copy(dinv_hbm.at[pl.ds(r0, rp)], dv)
        pltpu.sync_copy(dinvs_hbm.at[pl.ds(r0, rp)], dv2)
        pltpu.sync_copy(b_hbm, bv)

        def edge_loop():
            def body(i, carry):
                descs = []
                for bb in range(NBUF):
                    j = i * NBUF + bb
                    descs.append(pltpu.async_copy(
                        sp_p.at[idx_s.at[j]], rows.at[bb], gsem.at[bb]))
                sdescs = []
                for bb in range(NBUF):
                    j = i * NBUF + bb
                    descs[bb].wait()
                    sdescs.append(pltpu.async_copy(
                        rows.at[bb], sp_t.at[idx_d.at[j]], ssem.at[bb],
                        add=True))
                for bb in range(NBUF):
                    sdescs[bb].wait()
                return carry
            lax.fori_loop(0, nch // NBUF, body, 0)

        def rescale(scale_ref, final):
            if final:
                bq0 = bv[pl.ds(cid * fh, 16)]
                bq1 = bv[pl.ds(cid * fh + 16, 16)]
            for h in range(4):
                base = r0 + h * CR
                pltpu.sync_copy(sp_t.at[pl.ds(base, CR)], cbA)
                pltpu.sync_copy(sp_p.at[pl.ds(base, CR)], cbB)

                def rowbody(r, carry):
                    sc = plsc.load_gather(
                        scale_ref, [jnp.full((LANES,), h * CR + r, jnp.int32)])
                    for q in range(fh // LANES):
                        sl = pl.ds(q * LANES, LANES)
                        v = (cbA[r, sl] + cbB[r, sl]) * sc
                        if final:
                            v = v + (bq0 if q == 0 else bq1)
                        cbA[r, sl] = v
                    return carry
                lax.fori_loop(0, CR, rowbody, 0)
                if final:
                    pltpu.sync_copy(
                        cbA, out_hbm.at[pl.ds(base, CR), pl.ds(cid * fh, fh)])
                else:
                    pltpu.sync_copy(cbA, sp_p.at[pl.ds(base, CR)])
                    pltpu.sync_copy(z_hbm.at[pl.ds(h * CR, CR)],
                                    sp_t.at[pl.ds(base, CR)])

        plsc.subcore_barrier()
        edge_loop()
        plsc.subcore_barrier()
        rescale(dv, final=False)
        plsc.subcore_barrier()
        edge_loop()
        plsc.subcore_barrier()
        rescale(dv2, final=True)

    return maink


def _tc_prep_body(x_ref, w_ref, hist_ref, p0_ref, dinv_ref, dinvs_ref,
                  *, n, n16, fh):
    deg = jnp.sum(hist_ref[...], axis=0) + 1.0
    dinv_ref[...] = 1.0 / deg
    dinvs_ref[...] = lax.rsqrt(deg)
    xw = lax.dot_general(x_ref[...], w_ref[...],
                         (((1,), (1,)), ((), ())),
                         preferred_element_type=jnp.float32)
    xws = xw * dinvs_ref[pl.ds(0, n)][:, None]
    zpad = jnp.zeros((n16 - n, fh), jnp.float32)
    p0_ref[0] = jnp.concatenate([xws[:, :fh], zpad], axis=0)
    p0_ref[1] = jnp.concatenate([xws[:, fh:], zpad], axis=0)


def kernel(x, edge_index, W, b):
    n, f_in = x.shape
    fo = W.shape[0]
    fh = fo // 2
    e = edge_index.shape[1]

    n16 = NS * 8 * _cdiv(n + 1, NS * 8)

    src = edge_index[0]
    dst = edge_index[1]

    epw1 = CHUNK * _cdiv(_cdiv(e, NW), CHUNK)
    dst_p1 = jnp.concatenate(
        [dst, jnp.full((NW * epw1 - e,), n, jnp.int32)])

    nch = NBUF * _cdiv(_cdiv(e, NS), CHUNK * NBUF)
    epw2 = nch * CHUNK
    src_p2 = jnp.concatenate([src, jnp.zeros((NS * epw2 - e,), jnp.int32)])
    dst_p2 = jnp.concatenate([dst, jnp.full((NS * epw2 - e,), n, jnp.int32)])
    src3 = src_p2.reshape(NS, nch, CHUNK)
    dst3 = dst_p2.reshape(NS, nch, CHUNK)

    rp = n16 // NS
    zrows = jnp.zeros((rp, fh), jnp.float32)

    hist = _deg_kernel(n16, epw1, NW)(dst_p1)

    tc_prep = pl.pallas_call(
        functools.partial(_tc_prep_body, n=n, n16=n16, fh=fh),
        out_shape=(
            jax.ShapeDtypeStruct((2, n16, fh), jnp.float32),
            jax.ShapeDtypeStruct((n16,), jnp.float32),
            jax.ShapeDtypeStruct((n16,), jnp.float32),
        ),
    )
    p0, dinv, dinvs = tc_prep(x, W, hist)

    out = _main_kernel(n16, fh, nch)(p0, src3, dst3, zrows, dinv, dinvs, b)
    return out[:n]
